# Initial kernel scaffold; baseline (speedup 1.0000x reference)
import jax
import jax.numpy as jnp
from jax.experimental import pallas as pl


def kernel(angles, sin_table, cos_table):
    raise NotImplementedError("write your pallas kernel here")


# speedup vs baseline: 255.4878x; 255.4878x over previous
import functools
import math

import jax
import jax.numpy as jnp
from jax import lax
from jax.experimental import pallas as pl
from jax.experimental.pallas import tpu as pltpu
from jax.experimental.pallas import tpu_sc as plsc

RESOLUTION = 1024
TWO_PI = 2.0 * math.pi

NC = 2
NS = 16
L = 16
NW = NC * NS

N_TOTAL = 16384 * 200
PER_W = N_TOTAL // NW
CHUNK = 25600
NCHUNK = PER_W // CHUNK


def _phase_body(ang_hbm, sin_t_hbm, cos_t_hbm, sin_out_hbm, cos_out_hbm,
                ang_v, sin_v, cos_v, sin_tab, cos_tab):
    wid = lax.axis_index("s") * NC + lax.axis_index("c")
    base = wid * PER_W
    pltpu.sync_copy(sin_t_hbm, sin_tab)
    pltpu.sync_copy(cos_t_hbm, cos_tab)
    scale = jnp.float32((RESOLUTION - 1) / TWO_PI)

    for chunk in range(NCHUNK):
        off = base + chunk * CHUNK
        pltpu.sync_copy(ang_hbm.at[pl.ds(off, CHUNK)], ang_v)

        def inner(i, carry):
            sl = pl.ds(i * L, L)
            a = ang_v[sl]
            idx = (a * scale).astype(jnp.int32)
            sin_v[sl] = plsc.load_gather(sin_tab, [idx])
            cos_v[sl] = plsc.load_gather(cos_tab, [idx])
            return carry

        lax.fori_loop(0, CHUNK // L, inner, 0)

        pltpu.sync_copy(sin_v, sin_out_hbm.at[pl.ds(off, CHUNK)])
        pltpu.sync_copy(cos_v, cos_out_hbm.at[pl.ds(off, CHUNK)])


@jax.jit
def kernel(angles, sin_table, cos_table):
    shape = angles.shape
    flat = angles.reshape(-1)
    mesh = plsc.VectorSubcoreMesh(core_axis_name="c", subcore_axis_name="s")
    run = pl.kernel(
        _phase_body,
        out_type=(
            jax.ShapeDtypeStruct((N_TOTAL,), jnp.float32),
            jax.ShapeDtypeStruct((N_TOTAL,), jnp.float32),
        ),
        mesh=mesh,
        scratch_types=[
            pltpu.VMEM((CHUNK,), jnp.float32),
            pltpu.VMEM((CHUNK,), jnp.float32),
            pltpu.VMEM((CHUNK,), jnp.float32),
            pltpu.VMEM((RESOLUTION,), jnp.float32),
            pltpu.VMEM((RESOLUTION,), jnp.float32),
        ],
        compiler_params=pltpu.CompilerParams(needs_layout_passes=False),
    )
    sin_flat, cos_flat = run(flat, sin_table, cos_table)
    return sin_flat.reshape(shape), cos_flat.reshape(shape)


# speedup vs baseline: 326.1813x; 1.2767x over previous
import functools
import math

import jax
import jax.numpy as jnp
from jax import lax
from jax.experimental import pallas as pl
from jax.experimental.pallas import tpu as pltpu
from jax.experimental.pallas import tpu_sc as plsc

RESOLUTION = 1024
TWO_PI = 2.0 * math.pi

NC = 2
NS = 16
L = 16
NW = NC * NS

N_TOTAL = 16384 * 200
PER_W = N_TOTAL // NW
CHUNK = 25600
NCHUNK = PER_W // CHUNK


def _phase_body(ang_hbm, sin_t_hbm, cos_t_hbm, sin_out_hbm, cos_out_hbm,
                ang_v, sin_v, cos_v, sin_tab, cos_tab):
    wid = lax.axis_index("s") * NC + lax.axis_index("c")
    base = wid * PER_W
    pltpu.sync_copy(sin_t_hbm, sin_tab)
    pltpu.sync_copy(cos_t_hbm, cos_tab)
    scale = jnp.float32((RESOLUTION - 1) / TWO_PI)

    for chunk in range(NCHUNK):
        off = base + chunk * CHUNK
        pltpu.sync_copy(ang_hbm.at[pl.ds(off, CHUNK)], ang_v)

        @plsc.parallel_loop(0, CHUNK // L, unroll=8)
        def _(i):
            sl = pl.ds(i * L, L)
            a = ang_v[sl]
            idx = (a * scale).astype(jnp.int32)
            sin_v[sl] = plsc.load_gather(sin_tab, [idx])
            cos_v[sl] = plsc.load_gather(cos_tab, [idx])

        pltpu.sync_copy(sin_v, sin_out_hbm.at[pl.ds(off, CHUNK)])
        pltpu.sync_copy(cos_v, cos_out_hbm.at[pl.ds(off, CHUNK)])


@jax.jit
def kernel(angles, sin_table, cos_table):
    shape = angles.shape
    flat = angles.reshape(-1)
    mesh = plsc.VectorSubcoreMesh(core_axis_name="c", subcore_axis_name="s")
    run = pl.kernel(
        _phase_body,
        out_type=(
            jax.ShapeDtypeStruct((N_TOTAL,), jnp.float32),
            jax.ShapeDtypeStruct((N_TOTAL,), jnp.float32),
        ),
        mesh=mesh,
        scratch_types=[
            pltpu.VMEM((CHUNK,), jnp.float32),
            pltpu.VMEM((CHUNK,), jnp.float32),
            pltpu.VMEM((CHUNK,), jnp.float32),
            pltpu.VMEM((RESOLUTION,), jnp.float32),
            pltpu.VMEM((RESOLUTION,), jnp.float32),
        ],
        compiler_params=pltpu.CompilerParams(needs_layout_passes=False),
    )
    sin_flat, cos_flat = run(flat, sin_table, cos_table)
    return sin_flat.reshape(shape), cos_flat.reshape(shape)


# speedup vs baseline: 326.6528x; 1.0014x over previous
import functools
import math

import jax
import jax.numpy as jnp
from jax import lax
from jax.experimental import pallas as pl
from jax.experimental.pallas import tpu as pltpu
from jax.experimental.pallas import tpu_sc as plsc

RESOLUTION = 1024
TWO_PI = 2.0 * math.pi

NC = 2
NS = 16
L = 16
NW = NC * NS

N_TOTAL = 16384 * 200
PER_W = N_TOTAL // NW
CHUNK = 25600
NCHUNK = PER_W // CHUNK


def _phase_body(ang_hbm, sin_t_hbm, cos_t_hbm, sin_out_hbm, cos_out_hbm,
                ang_v, sin_v, cos_v, sin_tab, cos_tab):
    wid = lax.axis_index("s") * NC + lax.axis_index("c")
    base = wid * PER_W
    pltpu.sync_copy(sin_t_hbm, sin_tab)
    pltpu.sync_copy(cos_t_hbm, cos_tab)
    scale = jnp.float32((RESOLUTION - 1) / TWO_PI)

    for chunk in range(NCHUNK):
        off = base + chunk * CHUNK
        pltpu.sync_copy(ang_hbm.at[pl.ds(off, CHUNK)], ang_v)

        @plsc.parallel_loop(0, CHUNK // L, unroll=16)
        def _(i):
            sl = pl.ds(i * L, L)
            a = ang_v[sl]
            idx = (a * scale).astype(jnp.int32)
            sin_v[sl] = plsc.load_gather(sin_tab, [idx])
            cos_v[sl] = plsc.load_gather(cos_tab, [idx])

        pltpu.sync_copy(sin_v, sin_out_hbm.at[pl.ds(off, CHUNK)])
        pltpu.sync_copy(cos_v, cos_out_hbm.at[pl.ds(off, CHUNK)])


@jax.jit
def kernel(angles, sin_table, cos_table):
    shape = angles.shape
    flat = angles.reshape(-1)
    mesh = plsc.VectorSubcoreMesh(core_axis_name="c", subcore_axis_name="s")
    run = pl.kernel(
        _phase_body,
        out_type=(
            jax.ShapeDtypeStruct((N_TOTAL,), jnp.float32),
            jax.ShapeDtypeStruct((N_TOTAL,), jnp.float32),
        ),
        mesh=mesh,
        scratch_types=[
            pltpu.VMEM((CHUNK,), jnp.float32),
            pltpu.VMEM((CHUNK,), jnp.float32),
            pltpu.VMEM((CHUNK,), jnp.float32),
            pltpu.VMEM((RESOLUTION,), jnp.float32),
            pltpu.VMEM((RESOLUTION,), jnp.float32),
        ],
        compiler_params=pltpu.CompilerParams(needs_layout_passes=False),
    )
    sin_flat, cos_flat = run(flat, sin_table, cos_table)
    return sin_flat.reshape(shape), cos_flat.reshape(shape)


# speedup vs baseline: 349.4461x; 1.0698x over previous
import math

import jax
import jax.numpy as jnp
from jax import lax
from jax.experimental import pallas as pl
from jax.experimental.pallas import tpu as pltpu
from jax.experimental.pallas import tpu_sc as plsc

RESOLUTION = 1024
TWO_PI = 2.0 * math.pi

NC = 2
NS = 16
L = 16
NW = NC * NS

N_TOTAL = 16384 * 200
PER_W = N_TOTAL // NW
CHUNK = 6400
NCH = PER_W // CHUNK
NB = 4


def _phase_body(ang_hbm, sin_t_hbm, cos_t_hbm, sin_out_hbm, cos_out_hbm,
                *scratch):
    ang_v = scratch[0:NB]
    sin_v = scratch[NB:2 * NB]
    cos_v = scratch[2 * NB:3 * NB]
    sin_tab, cos_tab = scratch[3 * NB], scratch[3 * NB + 1]
    in_sems = scratch[3 * NB + 2:3 * NB + 2 + NB]
    s_sems = scratch[3 * NB + 2 + NB:3 * NB + 2 + 2 * NB]
    c_sems = scratch[3 * NB + 2 + 2 * NB:3 * NB + 2 + 3 * NB]

    wid = lax.axis_index("s") * NC + lax.axis_index("c")
    base = wid * PER_W
    scale = jnp.float32((RESOLUTION - 1) / TWO_PI)

    pltpu.sync_copy(sin_t_hbm, sin_tab)
    pltpu.sync_copy(cos_t_hbm, cos_tab)

    in_cp = [None] * NB
    s_cp = [None] * NB
    c_cp = [None] * NB

    for b in range(NB):
        off = base + b * CHUNK
        in_cp[b] = pltpu.async_copy(
            ang_hbm.at[pl.ds(off, CHUNK)], ang_v[b], in_sems[b])

    for c in range(NCH):
        b = c % NB
        in_cp[b].wait()
        if s_cp[b] is not None:
            s_cp[b].wait()
            c_cp[b].wait()

        av, sv, cv = ang_v[b], sin_v[b], cos_v[b]

        @plsc.parallel_loop(0, CHUNK // L, unroll=8)
        def _(i):
            sl = pl.ds(i * L, L)
            a = av[sl]
            idx = (a * scale).astype(jnp.int32)
            sv[sl] = plsc.load_gather(sin_tab, [idx])
            cv[sl] = plsc.load_gather(cos_tab, [idx])

        off = base + c * CHUNK
        s_cp[b] = pltpu.async_copy(
            sin_v[b], sin_out_hbm.at[pl.ds(off, CHUNK)], s_sems[b])
        c_cp[b] = pltpu.async_copy(
            cos_v[b], cos_out_hbm.at[pl.ds(off, CHUNK)], c_sems[b])

        nxt = c + NB
        if nxt < NCH:
            noff = base + nxt * CHUNK
            in_cp[b] = pltpu.async_copy(
                ang_hbm.at[pl.ds(noff, CHUNK)], ang_v[b], in_sems[b])

    for b in range(NB):
        s_cp[b].wait()
        c_cp[b].wait()


@jax.jit
def kernel(angles, sin_table, cos_table):
    shape = angles.shape
    flat = angles.reshape(-1)
    mesh = plsc.VectorSubcoreMesh(core_axis_name="c", subcore_axis_name="s")
    run = pl.kernel(
        _phase_body,
        out_type=(
            jax.ShapeDtypeStruct((N_TOTAL,), jnp.float32),
            jax.ShapeDtypeStruct((N_TOTAL,), jnp.float32),
        ),
        mesh=mesh,
        scratch_types=(
            [pltpu.VMEM((CHUNK,), jnp.float32) for _ in range(3 * NB)]
            + [pltpu.VMEM((RESOLUTION,), jnp.float32) for _ in range(2)]
            + [pltpu.SemaphoreType.DMA for _ in range(3 * NB)]
        ),
        compiler_params=pltpu.CompilerParams(needs_layout_passes=False),
    )
    sin_flat, cos_flat = run(flat, sin_table, cos_table)
    return sin_flat.reshape(shape), cos_flat.reshape(shape)


# speedup vs baseline: 579.0475x; 1.6570x over previous
import math

import jax
import jax.numpy as jnp
from jax import lax
from jax.experimental import pallas as pl
from jax.experimental.pallas import tpu as pltpu
from jax.experimental.pallas import tpu_sc as plsc

RESOLUTION = 1024
TWO_PI = 2.0 * math.pi

NC = 2
NS = 16
L = 16
NW = NC * NS

N_ROWS = 16384
N_COLS = 200
ROWS_PER_W = N_ROWS // NW
ROWS_PER_CHUNK = 32
NCH = ROWS_PER_W // ROWS_PER_CHUNK
NB = 4

COL_OFFS = tuple(16 * j for j in range(N_COLS // L)) + (N_COLS - L,)


def _phase_body(ang_hbm, sin_t_hbm, cos_t_hbm, sin_out_hbm, cos_out_hbm,
                *scratch):
    ang_v = scratch[0:NB]
    sin_v = scratch[NB:2 * NB]
    cos_v = scratch[2 * NB:3 * NB]
    sin_tab, cos_tab = scratch[3 * NB], scratch[3 * NB + 1]
    in_sems = scratch[3 * NB + 2:3 * NB + 2 + NB]
    s_sems = scratch[3 * NB + 2 + NB:3 * NB + 2 + 2 * NB]
    c_sems = scratch[3 * NB + 2 + 2 * NB:3 * NB + 2 + 3 * NB]

    wid = lax.axis_index("s") * NC + lax.axis_index("c")
    row_base = wid * ROWS_PER_W
    scale = jnp.float32((RESOLUTION - 1) / TWO_PI)

    pltpu.sync_copy(sin_t_hbm, sin_tab)
    pltpu.sync_copy(cos_t_hbm, cos_tab)

    in_cp = [None] * NB
    s_cp = [None] * NB
    c_cp = [None] * NB

    for b in range(NB):
        r0 = row_base + b * ROWS_PER_CHUNK
        in_cp[b] = pltpu.async_copy(
            ang_hbm.at[pl.ds(r0, ROWS_PER_CHUNK), :], ang_v[b], in_sems[b])

    for c in range(NCH):
        b = c % NB
        in_cp[b].wait()
        if s_cp[b] is not None:
            s_cp[b].wait()
            c_cp[b].wait()

        av, sv, cv = ang_v[b], sin_v[b], cos_v[b]

        @plsc.parallel_loop(0, ROWS_PER_CHUNK)
        def _(r):
            for off in COL_OFFS:
                sl = pl.ds(off, L)
                a = av[r, sl]
                idx = (a * scale).astype(jnp.int32)
                sv[r, sl] = plsc.load_gather(sin_tab, [idx])
                cv[r, sl] = plsc.load_gather(cos_tab, [idx])

        r0 = row_base + c * ROWS_PER_CHUNK
        s_cp[b] = pltpu.async_copy(
            sin_v[b], sin_out_hbm.at[pl.ds(r0, ROWS_PER_CHUNK), :], s_sems[b])
        c_cp[b] = pltpu.async_copy(
            cos_v[b], cos_out_hbm.at[pl.ds(r0, ROWS_PER_CHUNK), :], c_sems[b])

        nxt = c + NB
        if nxt < NCH:
            nr0 = row_base + nxt * ROWS_PER_CHUNK
            in_cp[b] = pltpu.async_copy(
                ang_hbm.at[pl.ds(nr0, ROWS_PER_CHUNK), :], ang_v[b],
                in_sems[b])

    for b in range(NB):
        s_cp[b].wait()
        c_cp[b].wait()


@jax.jit
def kernel(angles, sin_table, cos_table):
    mesh = plsc.VectorSubcoreMesh(core_axis_name="c", subcore_axis_name="s")
    run = pl.kernel(
        _phase_body,
        out_type=(
            jax.ShapeDtypeStruct((N_ROWS, N_COLS), jnp.float32),
            jax.ShapeDtypeStruct((N_ROWS, N_COLS), jnp.float32),
        ),
        mesh=mesh,
        scratch_types=(
            [pltpu.VMEM((ROWS_PER_CHUNK, N_COLS), jnp.float32)
             for _ in range(3 * NB)]
            + [pltpu.VMEM((RESOLUTION,), jnp.float32) for _ in range(2)]
            + [pltpu.SemaphoreType.DMA for _ in range(3 * NB)]
        ),
        compiler_params=pltpu.CompilerParams(needs_layout_passes=False),
    )
    return run(angles, sin_table, cos_table)
